# baseline (speedup 1.0000x reference)
import functools

import jax
import jax.numpy as jnp
from jax.experimental import pallas as pl
from jax.experimental.pallas import tpu as pltpu

_K = 8192
_D = 256
_B = 1024
_L = 16
_DECAY = 0.9
_THRESH = 1e-4

_BB = 128


def _pursuit_kernel(t_ref, cb_ref, idx_ref, recon_ref):
    residual = t_ref[...]
    cb = cb_ref[...]
    recon = jnp.zeros_like(residual)
    iota_k = jax.lax.broadcasted_iota(jnp.int32, (_BB, _K), 1)
    idx_cols = []
    for step in range(_L):
        rn = jnp.sqrt(jnp.sum(residual * residual, axis=1, keepdims=True))
        active = (rn > _THRESH).astype(jnp.float32)
        rnorm = residual / jnp.maximum(rn, 1e-8)
        cos = jax.lax.dot_general(
            rnorm, cb, (((1,), (1,)), ((), ())),
            preferred_element_type=jnp.float32,
            precision=jax.lax.Precision.DEFAULT)
        best = jnp.argmax(jnp.abs(cos), axis=1)
        onehot_b = iota_k == best[:, None]
        best_cos = jnp.sum(jnp.where(onehot_b, cos, 0.0), axis=1)
        sign = jnp.where(best_cos >= 0, 1.0, -1.0)
        signed_idx = jnp.where(best_cos >= 0, best, -(best + 1))
        decay = _DECAY ** (step + 1)
        row = jax.lax.dot_general(
            onehot_b.astype(jnp.float32), cb, (((1,), (0,)), ((), ())),
            preferred_element_type=jnp.float32,
            precision=jax.lax.Precision.HIGHEST)
        contribution = (active * sign[:, None]) * (decay * row)
        residual = residual - contribution
        recon = recon + contribution
        idx_cols.append(signed_idx.astype(jnp.int32))
    idx_ref[...] = jnp.stack(idx_cols, axis=1)
    recon_ref[...] = recon


@jax.jit
def kernel(targets, codebook):
    idx, recon = pl.pallas_call(
        _pursuit_kernel,
        grid=(_B // _BB,),
        in_specs=[
            pl.BlockSpec((_BB, _D), lambda i: (i, 0)),
            pl.BlockSpec((_K, _D), lambda i: (0, 0)),
        ],
        out_specs=[
            pl.BlockSpec((_BB, _L), lambda i: (i, 0)),
            pl.BlockSpec((_BB, _D), lambda i: (i, 0)),
        ],
        out_shape=[
            jax.ShapeDtypeStruct((_B, _L), jnp.int32),
            jax.ShapeDtypeStruct((_B, _D), jnp.float32),
        ],
        compiler_params=pltpu.CompilerParams(
            dimension_semantics=("arbitrary",),
        ),
    )(targets, codebook)
    return idx, recon


# speedup vs baseline: 2.8097x; 2.8097x over previous
import functools

import jax
import jax.numpy as jnp
from jax import lax
from jax.experimental import pallas as pl
from jax.experimental.pallas import tpu as pltpu
from jax.experimental.pallas import tpu_sc as plsc

_K = 8192
_D = 256
_B = 1024
_L = 16
_DECAY = 0.9
_THRESH = 1e-4

_BB = 128


def _tc_step_kernel(decay, apply_update, res_ref, rec_ref, row_ref, w_ref,
                    cb_ref, best_ref, sidx_ref, w_out_ref, res_out_ref,
                    rec_out_ref):
    residual = res_ref[...]
    recon = rec_ref[...]
    if apply_update:
        contribution = w_ref[...] * row_ref[...]
        residual = residual - contribution
        recon = recon + contribution
    cb = cb_ref[...]
    rn = jnp.sqrt(jnp.sum(residual * residual, axis=1, keepdims=True))
    active = (rn > _THRESH).astype(jnp.float32)
    rnorm = residual / jnp.maximum(rn, 1e-8)
    cos = lax.dot_general(
        rnorm, cb, (((1,), (1,)), ((), ())),
        preferred_element_type=jnp.float32,
        precision=lax.Precision.DEFAULT)
    maxpos = jnp.max(cos, axis=1)
    minneg = jnp.min(cos, axis=1)
    ipos = jnp.argmax(cos, axis=1).astype(jnp.int32)
    ineg = jnp.argmin(cos, axis=1).astype(jnp.int32)
    pos_wins = (maxpos > -minneg) | ((maxpos == -minneg) & (ipos < ineg))
    best = jnp.where(pos_wins, ipos, ineg)
    sign = jnp.where(pos_wins, 1.0, -1.0)
    signed_idx = jnp.where(pos_wins, best, -(best + 1))
    w = (active[:, 0] * sign) * decay
    best_ref[0, :] = best
    sidx_ref[0, :] = signed_idx
    w_out_ref[...] = w[:, None]
    res_out_ref[...] = residual
    rec_out_ref[...] = recon


def _tc_step(decay, apply_update, residual, recon, rows, w, codebook):
    row_spec = pl.BlockSpec((_BB, _D), lambda i: (i, 0))
    kern = functools.partial(_tc_step_kernel, decay, apply_update)
    return pl.pallas_call(
        kern,
        grid=(_B // _BB,),
        in_specs=[
            row_spec,
            row_spec,
            row_spec,
            pl.BlockSpec((_BB, 1), lambda i: (i, 0)),
            pl.BlockSpec((_K, _D), lambda i: (0, 0)),
        ],
        out_specs=[
            pl.BlockSpec((1, _BB), lambda i: (0, i)),
            pl.BlockSpec((1, _BB), lambda i: (0, i)),
            pl.BlockSpec((_BB, 1), lambda i: (i, 0)),
            row_spec,
            row_spec,
        ],
        out_shape=[
            jax.ShapeDtypeStruct((1, _B), jnp.int32),
            jax.ShapeDtypeStruct((1, _B), jnp.int32),
            jax.ShapeDtypeStruct((_B, 1), jnp.float32),
            jax.ShapeDtypeStruct((_B, _D), jnp.float32),
            jax.ShapeDtypeStruct((_B, _D), jnp.float32),
        ],
    )(residual, recon, rows, w, codebook)


def _tc_final_kernel(res_ref, rec_ref, row_ref, w_ref, rec_out_ref):
    contribution = w_ref[...] * row_ref[...]
    rec_out_ref[...] = rec_ref[...] + contribution
    del res_ref


def _tc_final(residual, recon, rows, w):
    row_spec = pl.BlockSpec((_BB, _D), lambda i: (i, 0))
    return pl.pallas_call(
        _tc_final_kernel,
        grid=(_B // _BB,),
        in_specs=[
            row_spec, row_spec, row_spec,
            pl.BlockSpec((_BB, 1), lambda i: (i, 0)),
        ],
        out_specs=row_spec,
        out_shape=jax.ShapeDtypeStruct((_B, _D), jnp.float32),
    )(residual, recon, rows, w)


_SC_NUM_CORES = 2
_SC_NUM_SUBCORES = 16


@functools.cache
def _make_sc_gather():
    nw = _SC_NUM_CORES * _SC_NUM_SUBCORES
    b_per_w = _B // nw
    mesh = plsc.VectorSubcoreMesh(core_axis_name="c", subcore_axis_name="s",
                                  num_cores=_SC_NUM_CORES)

    @functools.partial(
        pl.kernel, mesh=mesh,
        out_type=jax.ShapeDtypeStruct((_B, _D), jnp.float32),
        scratch_types=[
            pltpu.VMEM((b_per_w,), jnp.int32),
            pltpu.VMEM((b_per_w, _D), jnp.float32),
            pltpu.SemaphoreType.DMA,
        ],
    )
    def gather(table_hbm, idx_hbm, out_hbm, idx_v, rows_v, sem):
        wid = lax.axis_index("s") * _SC_NUM_CORES + lax.axis_index("c")
        base = wid * b_per_w
        pltpu.sync_copy(idx_hbm.at[pl.ds(base, b_per_w)], idx_v)
        pltpu.async_copy(table_hbm.at[idx_v], rows_v, sem).wait()
        pltpu.sync_copy(rows_v, out_hbm.at[pl.ds(base, b_per_w)])

    return gather


def _sc_gather(table, idx):
    return _make_sc_gather()(table, idx)


@jax.jit
def kernel(targets, codebook):
    residual = targets
    recon = jnp.zeros_like(targets)
    rows = jnp.zeros_like(targets)
    w = jnp.zeros((_B, 1), jnp.float32)
    idx_steps = []
    for step in range(_L):
        decay = _DECAY ** (step + 1)
        best, signed_idx, w, residual, recon = _tc_step(
            decay, step > 0, residual, recon, rows, w, codebook)
        idx_steps.append(signed_idx[0])
        rows = _sc_gather(codebook, best[0])
    recon = _tc_final(residual, recon, rows, w)
    signed_indices = jnp.stack(idx_steps, axis=1)
    return signed_indices, recon


# speedup vs baseline: 3.0713x; 1.0931x over previous
import functools

import jax
import jax.numpy as jnp
from jax import lax
from jax.experimental import pallas as pl
from jax.experimental.pallas import tpu as pltpu
from jax.experimental.pallas import tpu_sc as plsc

_K = 8192
_D = 256
_B = 1024
_L = 16
_DECAY = 0.9
_THRESH = 1e-4

_BB = 128
_BH = _B // 2

_SC_NUM_CORES = 2
_SC_NUM_SUBCORES = 16


def _tc_step_kernel(decay, apply_update, res_ref, rec_ref, row_ref, w_ref,
                    cb_ref, best_ref, sidx_ref, w_out_ref, res_out_ref,
                    rec_out_ref):
    residual = res_ref[...]
    recon = rec_ref[...]
    if apply_update:
        contribution = w_ref[...] * row_ref[...]
        residual = residual - contribution
        recon = recon + contribution
    rn = jnp.sqrt(jnp.sum(residual * residual, axis=1, keepdims=True))
    active = (rn > _THRESH).astype(jnp.float32)
    rnorm = residual / jnp.maximum(rn, 1e-8)
    cos = lax.dot_general(
        rnorm.astype(jnp.bfloat16), cb_ref[...], (((1,), (1,)), ((), ())),
        preferred_element_type=jnp.float32,
        precision=lax.Precision.DEFAULT)
    maxpos = jnp.max(cos, axis=1)
    minneg = jnp.min(cos, axis=1)
    ipos = jnp.argmax(cos, axis=1).astype(jnp.int32)
    ineg = jnp.argmin(cos, axis=1).astype(jnp.int32)
    pos_wins = (maxpos > -minneg) | ((maxpos == -minneg) & (ipos < ineg))
    best = jnp.where(pos_wins, ipos, ineg)
    sign = jnp.where(pos_wins, 1.0, -1.0)
    signed_idx = jnp.where(pos_wins, best, -(best + 1))
    w = (active[:, 0] * sign) * decay
    best_ref[0, :] = best
    sidx_ref[0, :] = signed_idx
    w_out_ref[...] = w[:, None]
    res_out_ref[...] = residual
    rec_out_ref[...] = recon


def _tc_step(decay, apply_update, residual, recon, rows, w, cb_bf16):
    row_spec = pl.BlockSpec((_BB, _D), lambda i: (i, 0))
    kern = functools.partial(_tc_step_kernel, decay, apply_update)
    return pl.pallas_call(
        kern,
        grid=(_BH // _BB,),
        in_specs=[
            row_spec,
            row_spec,
            row_spec,
            pl.BlockSpec((_BB, 1), lambda i: (i, 0)),
            pl.BlockSpec((_K, _D), lambda i: (0, 0)),
        ],
        out_specs=[
            pl.BlockSpec((1, _BB), lambda i: (0, i)),
            pl.BlockSpec((1, _BB), lambda i: (0, i)),
            pl.BlockSpec((_BB, 1), lambda i: (i, 0)),
            row_spec,
            row_spec,
        ],
        out_shape=[
            jax.ShapeDtypeStruct((1, _BH), jnp.int32),
            jax.ShapeDtypeStruct((1, _BH), jnp.int32),
            jax.ShapeDtypeStruct((_BH, 1), jnp.float32),
            jax.ShapeDtypeStruct((_BH, _D), jnp.float32),
            jax.ShapeDtypeStruct((_BH, _D), jnp.float32),
        ],
    )(residual, recon, rows, w, cb_bf16)


def _tc_final_kernel(rec_ref, row_ref, w_ref, rec_out_ref):
    rec_out_ref[...] = rec_ref[...] + w_ref[...] * row_ref[...]


def _tc_final(recon, rows, w):
    row_spec = pl.BlockSpec((_BB, _D), lambda i: (i, 0))
    return pl.pallas_call(
        _tc_final_kernel,
        grid=(_BH // _BB,),
        in_specs=[
            row_spec, row_spec,
            pl.BlockSpec((_BB, 1), lambda i: (i, 0)),
        ],
        out_specs=row_spec,
        out_shape=jax.ShapeDtypeStruct((_BH, _D), jnp.float32),
    )(recon, rows, w)


@functools.cache
def _make_sc_gather():
    nw = _SC_NUM_CORES * _SC_NUM_SUBCORES
    b_per_w = _BH // nw
    mesh = plsc.VectorSubcoreMesh(core_axis_name="c", subcore_axis_name="s",
                                  num_cores=_SC_NUM_CORES)

    @functools.partial(
        pl.kernel, mesh=mesh,
        out_type=jax.ShapeDtypeStruct((_BH, _D), jnp.float32),
        scratch_types=[
            pltpu.VMEM((b_per_w,), jnp.int32),
            pltpu.VMEM((b_per_w, _D), jnp.float32),
            pltpu.SemaphoreType.DMA,
        ],
    )
    def gather(table_hbm, idx_hbm, out_hbm, idx_v, rows_v, sem):
        wid = lax.axis_index("s") * _SC_NUM_CORES + lax.axis_index("c")
        base = wid * b_per_w
        pltpu.sync_copy(idx_hbm.at[pl.ds(base, b_per_w)], idx_v)
        pltpu.async_copy(table_hbm.at[idx_v], rows_v, sem).wait()
        pltpu.sync_copy(rows_v, out_hbm.at[pl.ds(base, b_per_w)])

    return gather


def _sc_gather(table, idx):
    return _make_sc_gather()(table, idx)


@jax.jit
def kernel(targets, codebook):
    cb_bf16 = codebook.astype(jnp.bfloat16)
    halves = []
    for h in range(2):
        halves.append({
            "residual": targets[h * _BH:(h + 1) * _BH],
            "recon": jnp.zeros((_BH, _D), jnp.float32),
            "rows": jnp.zeros((_BH, _D), jnp.float32),
            "w": jnp.zeros((_BH, 1), jnp.float32),
            "idx_steps": [],
        })
    for step in range(_L):
        decay = _DECAY ** (step + 1)
        for st in halves:
            best, signed_idx, st["w"], st["residual"], st["recon"] = _tc_step(
                decay, step > 0, st["residual"], st["recon"], st["rows"],
                st["w"], cb_bf16)
            st["idx_steps"].append(signed_idx[0])
            st["rows"] = _sc_gather(codebook, best[0])
    recons = [_tc_final(st["recon"], st["rows"], st["w"]) for st in halves]
    recon = jnp.concatenate(recons, axis=0)
    signed_indices = jnp.concatenate(
        [jnp.stack(st["idx_steps"], axis=1) for st in halves], axis=0)
    return signed_indices, recon


# speedup vs baseline: 3.3039x; 1.0757x over previous
import functools

import jax
import jax.numpy as jnp
from jax import lax
from jax.experimental import pallas as pl
from jax.experimental.pallas import tpu as pltpu
from jax.experimental.pallas import tpu_sc as plsc

_K = 8192
_D = 256
_B = 1024
_L = 16
_DECAY = 0.9
_THRESH = 1e-4

_BB = 256
_BH = _B // 2

_SC_NUM_CORES = 2
_SC_NUM_SUBCORES = 16


def _tc_step_kernel(decay, apply_update, res_ref, rec_ref, row_ref, w_ref,
                    cb_ref, best_ref, sidx_ref, w_out_ref, res_out_ref,
                    rec_out_ref):
    residual = res_ref[...]
    recon = rec_ref[...]
    if apply_update:
        contribution = w_ref[...] * row_ref[...]
        residual = residual - contribution
        recon = recon + contribution
    rn = jnp.sqrt(jnp.sum(residual * residual, axis=1, keepdims=True))
    active = (rn > _THRESH).astype(jnp.float32)
    rnorm = residual / jnp.maximum(rn, 1e-8)
    cos = lax.dot_general(
        rnorm.astype(jnp.bfloat16), cb_ref[...], (((1,), (1,)), ((), ())),
        preferred_element_type=jnp.float32,
        precision=lax.Precision.DEFAULT)
    maxpos = jnp.max(cos, axis=1)
    minneg = jnp.min(cos, axis=1)
    ipos = jnp.argmax(cos, axis=1).astype(jnp.int32)
    ineg = jnp.argmin(cos, axis=1).astype(jnp.int32)
    pos_wins = (maxpos > -minneg) | ((maxpos == -minneg) & (ipos < ineg))
    best = jnp.where(pos_wins, ipos, ineg)
    sign = jnp.where(pos_wins, 1.0, -1.0)
    signed_idx = jnp.where(pos_wins, best, -(best + 1))
    w = (active[:, 0] * sign) * decay
    best_ref[0, :] = best
    sidx_ref[0, :] = signed_idx
    w_out_ref[...] = w[:, None]
    res_out_ref[...] = residual
    rec_out_ref[...] = recon


def _tc_step(decay, apply_update, residual, recon, rows, w, cb_bf16):
    row_spec = pl.BlockSpec((_BB, _D), lambda i: (i, 0))
    kern = functools.partial(_tc_step_kernel, decay, apply_update)
    return pl.pallas_call(
        kern,
        grid=(_BH // _BB,),
        in_specs=[
            row_spec,
            row_spec,
            row_spec,
            pl.BlockSpec((_BB, 1), lambda i: (i, 0)),
            pl.BlockSpec((_K, _D), lambda i: (0, 0)),
        ],
        out_specs=[
            pl.BlockSpec((1, _BB), lambda i: (0, i)),
            pl.BlockSpec((1, _BB), lambda i: (0, i)),
            pl.BlockSpec((_BB, 1), lambda i: (i, 0)),
            row_spec,
            row_spec,
        ],
        out_shape=[
            jax.ShapeDtypeStruct((1, _BH), jnp.int32),
            jax.ShapeDtypeStruct((1, _BH), jnp.int32),
            jax.ShapeDtypeStruct((_BH, 1), jnp.float32),
            jax.ShapeDtypeStruct((_BH, _D), jnp.float32),
            jax.ShapeDtypeStruct((_BH, _D), jnp.float32),
        ],
    )(residual, recon, rows, w, cb_bf16)


def _tc_final_kernel(rec_ref, row_ref, w_ref, rec_out_ref):
    rec_out_ref[...] = rec_ref[...] + w_ref[...] * row_ref[...]


def _tc_final(recon, rows, w):
    row_spec = pl.BlockSpec((_BB, _D), lambda i: (i, 0))
    return pl.pallas_call(
        _tc_final_kernel,
        grid=(_BH // _BB,),
        in_specs=[
            row_spec, row_spec,
            pl.BlockSpec((_BB, 1), lambda i: (i, 0)),
        ],
        out_specs=row_spec,
        out_shape=jax.ShapeDtypeStruct((_BH, _D), jnp.float32),
    )(recon, rows, w)


@functools.cache
def _make_sc_gather():
    nw = _SC_NUM_CORES * _SC_NUM_SUBCORES
    b_per_w = _BH // nw
    mesh = plsc.VectorSubcoreMesh(core_axis_name="c", subcore_axis_name="s",
                                  num_cores=_SC_NUM_CORES)

    @functools.partial(
        pl.kernel, mesh=mesh,
        out_type=jax.ShapeDtypeStruct((_BH, _D), jnp.float32),
        scratch_types=[
            pltpu.VMEM((b_per_w,), jnp.int32),
            pltpu.VMEM((b_per_w, _D), jnp.float32),
            pltpu.SemaphoreType.DMA,
        ],
    )
    def gather(table_hbm, idx_hbm, out_hbm, idx_v, rows_v, sem):
        wid = lax.axis_index("s") * _SC_NUM_CORES + lax.axis_index("c")
        base = wid * b_per_w
        pltpu.sync_copy(idx_hbm.at[pl.ds(base, b_per_w)], idx_v)
        pltpu.async_copy(table_hbm.at[idx_v], rows_v, sem).wait()
        pltpu.sync_copy(rows_v, out_hbm.at[pl.ds(base, b_per_w)])

    return gather


def _sc_gather(table, idx):
    return _make_sc_gather()(table, idx)


@jax.jit
def kernel(targets, codebook):
    cb_bf16 = codebook.astype(jnp.bfloat16)
    halves = []
    for h in range(2):
        halves.append({
            "residual": targets[h * _BH:(h + 1) * _BH],
            "recon": jnp.zeros((_BH, _D), jnp.float32),
            "rows": jnp.zeros((_BH, _D), jnp.float32),
            "w": jnp.zeros((_BH, 1), jnp.float32),
            "idx_steps": [],
        })
    for step in range(_L):
        decay = _DECAY ** (step + 1)
        for st in halves:
            best, signed_idx, st["w"], st["residual"], st["recon"] = _tc_step(
                decay, step > 0, st["residual"], st["recon"], st["rows"],
                st["w"], cb_bf16)
            st["idx_steps"].append(signed_idx[0])
            st["rows"] = _sc_gather(codebook, best[0])
    recons = [_tc_final(st["recon"], st["rows"], st["w"]) for st in halves]
    recon = jnp.concatenate(recons, axis=0)
    signed_indices = jnp.concatenate(
        [jnp.stack(st["idx_steps"], axis=1) for st in halves], axis=0)
    return signed_indices, recon
